# baseline (speedup 1.0000x reference)
import functools

import jax
import jax.numpy as jnp
from jax import lax
from jax.experimental import pallas as pl
from jax.experimental.pallas import tpu as pltpu
from jax.experimental.pallas import tpu_sc as plsc

N_TYPES = 100
D = 512
B = 100000
NC = 2
NS = 16
NW = NC * NS
C = 64
NSLOT = 50
RPW = NSLOT * C
LAST_N = B - (NW - 1) * RPW
TAIL = LAST_N - (LAST_N // C) * C
L = 16


def _emb_body(idx_hbm, w_hbm, out_hbm, table_v, idx_v, rows0, rows1, tsem, o0, o1):
    wid = lax.axis_index("s") * NC + lax.axis_index("c")
    base = wid * RPW
    nval = jnp.where(wid == NW - 1, LAST_N // C, NSLOT)

    pltpu.async_copy(w_hbm, table_v, tsem)

    @pl.when(wid == NW - 1)
    def _():
        pltpu.sync_copy(idx_hbm.at[pl.ds(base, LAST_N)], idx_v.at[pl.ds(0, LAST_N)])

    @pl.when(wid != NW - 1)
    def _():
        pltpu.sync_copy(idx_hbm.at[pl.ds(base, RPW)], idx_v)

    pltpu.make_async_copy(w_hbm, table_v, tsem).wait()

    def copy_row(t, rows, r):
        for cb in range(0, D, L * 16):
            vals = [table_v[t, pl.ds(cb + k * L, L)] for k in range(16)]
            for k in range(16):
                rows[r, pl.ds(cb + k * L, L)] = vals[k]

    def compute(j, rows):
        @plsc.parallel_loop(0, C, unroll=2)
        def _(r):
            t = idx_v[pl.ds(j * C + r, L)][0]
            copy_row(t, rows, r)

    def scatter_start(j, rows, sem):
        pltpu.async_copy(rows, out_hbm.at[pl.ds(base + j * C, C)], sem)

    def scatter_wait(rows, sem):
        pltpu.make_async_copy(rows, out_hbm.at[pl.ds(base, C)], sem).wait()

    def step(t, carry):
        j0 = 2 * t
        j1 = j0 + 1

        @pl.when(t > 0)
        def _():
            scatter_wait(rows0, o0)

        compute(j0, rows0)
        scatter_start(j0, rows0, o0)

        @pl.when(t > 0)
        def _():
            scatter_wait(rows1, o1)

        compute(j1, rows1)
        scatter_start(j1, rows1, o1)
        return carry

    lax.fori_loop(0, nval // 2, step, 0)
    scatter_wait(rows0, o0)
    scatter_wait(rows1, o1)

    @pl.when(wid == NW - 1)
    def _():
        tbase = (LAST_N // C) * C
        for g in range(TAIL // L):
            idx16 = idx_v[pl.ds(tbase + g * L, L)]
            for lane in range(L):
                copy_row(idx16[lane], rows0, g * L + lane)
        pltpu.async_copy(
            rows0.at[pl.ds(0, TAIL)],
            out_hbm.at[pl.ds(base + tbase, TAIL)],
            o0,
        )
        pltpu.make_async_copy(
            rows0.at[pl.ds(0, TAIL)],
            out_hbm.at[pl.ds(base + tbase, TAIL)],
            o0,
        ).wait()


@jax.jit
def _emb(idx, w):
    mesh = plsc.VectorSubcoreMesh(core_axis_name="c", subcore_axis_name="s")
    f = functools.partial(
        pl.kernel,
        mesh=mesh,
        out_type=jax.ShapeDtypeStruct((B, D), jnp.float32),
        scratch_types=[
            pltpu.VMEM((N_TYPES, D), jnp.float32),
            pltpu.VMEM((RPW,), jnp.int32),
            pltpu.VMEM((C, D), jnp.float32),
            pltpu.VMEM((C, D), jnp.float32),
            pltpu.SemaphoreType.DMA,
            pltpu.SemaphoreType.DMA,
            pltpu.SemaphoreType.DMA,
        ],
    )(_emb_body)
    return f(idx, w)


def kernel(atom_numbers, W):
    idx = jnp.squeeze(atom_numbers, axis=-1)
    return _emb(idx, W)


# speedup vs baseline: 1.0704x; 1.0704x over previous
import functools

import jax
import jax.numpy as jnp
from jax import lax
from jax.experimental import pallas as pl
from jax.experimental.pallas import tpu as pltpu
from jax.experimental.pallas import tpu_sc as plsc

N_TYPES = 100
D = 512
B = 100000
NC = 2
NS = 16
NW = NC * NS
RPW = 3200
LAST_N = B - (NW - 1) * RPW
L = 16
K = 8


def _emb_body(idx_hbm, w_hbm, out_hbm, table_v, idx_v, tsem, osem):
    wid = lax.axis_index("s") * NC + lax.axis_index("c")
    base = wid * RPW
    n = jnp.where(wid == NW - 1, LAST_N, RPW)

    pltpu.async_copy(w_hbm, table_v, tsem)

    @pl.when(wid == NW - 1)
    def _():
        pltpu.sync_copy(idx_hbm.at[pl.ds(base, LAST_N)], idx_v.at[pl.ds(0, LAST_N)])

    @pl.when(wid != NW - 1)
    def _():
        pltpu.sync_copy(idx_hbm.at[pl.ds(base, RPW)], idx_v.at[pl.ds(0, RPW)])

    pltpu.make_async_copy(w_hbm, table_v, tsem).wait()

    def issue(i):
        t = idx_v[pl.ds(i, L)][0]
        pltpu.async_copy(table_v.at[pl.ds(t, 1)], out_hbm.at[pl.ds(base + i, 1)], osem)

    def wait_one():
        pltpu.make_async_copy(
            table_v.at[pl.ds(0, 1)], out_hbm.at[pl.ds(base, 1)], osem
        ).wait()

    for i in range(K):
        issue(i)

    def step(i, carry):
        wait_one()
        issue(i)
        return carry

    lax.fori_loop(K, n, step, 0)
    for _ in range(K):
        wait_one()


@jax.jit
def _emb(idx, w):
    mesh = plsc.VectorSubcoreMesh(core_axis_name="c", subcore_axis_name="s")
    f = functools.partial(
        pl.kernel,
        mesh=mesh,
        out_type=jax.ShapeDtypeStruct((B, D), jnp.float32),
        scratch_types=[
            pltpu.VMEM((N_TYPES, D), jnp.float32),
            pltpu.VMEM((RPW + L,), jnp.int32),
            pltpu.SemaphoreType.DMA,
            pltpu.SemaphoreType.DMA,
        ],
    )(_emb_body)
    return f(idx, w)


def kernel(atom_numbers, W):
    idx = jnp.squeeze(atom_numbers, axis=-1)
    return _emb(idx, W)
